# Initial kernel scaffold; baseline (speedup 1.0000x reference)
import functools

import jax
import jax.numpy as jnp
from jax import lax
from jax.experimental import pallas as pl
from jax.experimental.pallas import tpu as pltpu
from jax.experimental.pallas import tpu_sc as plsc

NUM_WORKERS = 32
CHUNK = 128
LANES = 16


@functools.partial(jax.jit, static_argnums=(2, 3, 4))
def _sc_embed(idx, table, B, K, N):
    D = table.shape[1]
    chunks_per_batch = N // CHUNK
    batches_per_worker = B // NUM_WORKERS
    total_chunks = batches_per_worker * chunks_per_batch

    mesh = plsc.VectorSubcoreMesh(core_axis_name="c", subcore_axis_name="s")

    @functools.partial(
        pl.kernel,
        out_type=jax.ShapeDtypeStruct((B, D, N), jnp.float32),
        mesh=mesh,
        scratch_types=[
            pltpu.VMEM((K, CHUNK), jnp.int32),
            pltpu.VMEM((K, CHUNK, D), jnp.float32),
            pltpu.VMEM((D, CHUNK), jnp.float32),
            pltpu.SemaphoreType.DMA,
        ],
    )
    def body(idx_hbm, table_hbm, out_hbm, idx_v, rows_v, acc_t, sem):
        wid = lax.axis_index("s") * 2 + lax.axis_index("c")
        iota = lax.iota(jnp.int32, LANES)

        def chunk_step(i, _):
            b = wid * batches_per_worker + (i // chunks_per_batch)
            base = (i % chunks_per_batch) * CHUNK
            pltpu.sync_copy(idx_hbm.at[b, :, pl.ds(base, CHUNK)], idx_v)
            copies = [
                pltpu.async_copy(table_hbm.at[idx_v.at[k]], rows_v.at[k], sem)
                for k in range(K)
            ]
            for cp in copies:
                cp.wait()

            def pos_step(p, _):
                pvec = jnp.full((LANES,), p, jnp.int32)
                for half in range(D // LANES):
                    acc = rows_v[0, p, pl.ds(half * LANES, LANES)]
                    for k in range(1, K):
                        acc = acc + rows_v[k, p, pl.ds(half * LANES, LANES)]
                    acc = acc * (1.0 / K)
                    dvec = iota + (half * LANES)
                    plsc.store_scatter(acc_t, [dvec, pvec], acc)
                return 0

            lax.fori_loop(0, CHUNK, pos_step, 0, unroll=False)
            pltpu.sync_copy(acc_t, out_hbm.at[b, :, pl.ds(base, CHUNK)])
            return 0

        lax.fori_loop(0, total_chunks, chunk_step, 0, unroll=False)

    return body(idx, table)


def kernel(inputs, table):
    B, K, H, W = inputs.shape
    N = H * W
    idx = inputs.reshape(B, K, N).astype(jnp.int32)
    out = _sc_embed(idx, table, B, K, N)
    return out.reshape(B, table.shape[1], H, W)


# speedup vs baseline: 10.9026x; 10.9026x over previous
import functools

import jax
import jax.numpy as jnp
from jax import lax
from jax.experimental import pallas as pl
from jax.experimental.pallas import tpu as pltpu
from jax.experimental.pallas import tpu_sc as plsc

NUM_WORKERS = 32
CHUNK = 128
LANES = 16


@functools.partial(jax.jit, static_argnums=(2, 3, 4))
def _sc_embed(idx, table, B, K, N):
    D = table.shape[1]
    chunks_per_batch = N // CHUNK
    batches_per_worker = B // NUM_WORKERS
    total_chunks = batches_per_worker * chunks_per_batch

    mesh = plsc.VectorSubcoreMesh(core_axis_name="c", subcore_axis_name="s")

    @functools.partial(
        pl.kernel,
        out_type=jax.ShapeDtypeStruct((B, N, D), jnp.float32),
        mesh=mesh,
        scratch_types=[
            pltpu.VMEM((K, CHUNK), jnp.int32),
            pltpu.VMEM((K, CHUNK, D), jnp.float32),
            pltpu.VMEM((CHUNK, D), jnp.float32),
            pltpu.SemaphoreType.DMA,
        ],
        compiler_params=pltpu.CompilerParams(use_tc_tiling_on_sc=False),
    )
    def body(idx_hbm, table_hbm, out_hbm, idx_v, rows_v, acc_t, sem):
        wid = lax.axis_index("s") * 2 + lax.axis_index("c")

        def chunk_step(i, _):
            b = wid * batches_per_worker + (i // chunks_per_batch)
            base = (i % chunks_per_batch) * CHUNK
            pltpu.sync_copy(idx_hbm.at[b, :, pl.ds(base, CHUNK)], idx_v)
            copies = [
                pltpu.async_copy(table_hbm.at[idx_v.at[k]], rows_v.at[k], sem)
                for k in range(K)
            ]
            for cp in copies:
                cp.wait()

            def pos_step(p, _):
                for half in range(D // LANES):
                    acc = rows_v[0, p, pl.ds(half * LANES, LANES)]
                    for k in range(1, K):
                        acc = acc + rows_v[k, p, pl.ds(half * LANES, LANES)]
                    acc_t[p, pl.ds(half * LANES, LANES)] = acc * (1.0 / K)
                return 0

            lax.fori_loop(0, CHUNK, pos_step, 0, unroll=False)
            pltpu.sync_copy(acc_t, out_hbm.at[b, pl.ds(base, CHUNK), :])
            return 0

        lax.fori_loop(0, total_chunks, chunk_step, 0, unroll=False)

    return body(idx, table)


def _transpose_block(x_ref, o_ref):
    o_ref[...] = jnp.swapaxes(x_ref[...], 1, 2)


@functools.partial(jax.jit, static_argnums=(1, 2, 3))
def _tc_transpose(x, B, N, D):
    NB = 512
    return pl.pallas_call(
        _transpose_block,
        grid=(B, N // NB),
        in_specs=[pl.BlockSpec((1, NB, D), lambda b, n: (b, n, 0))],
        out_specs=pl.BlockSpec((1, D, NB), lambda b, n: (b, 0, n)),
        out_shape=jax.ShapeDtypeStruct((B, D, N), jnp.float32),
    )(x)


def kernel(inputs, table):
    B, K, H, W = inputs.shape
    N = H * W
    D = table.shape[1]
    idx = inputs.reshape(B, K, N).astype(jnp.int32)
    pm = _sc_embed(idx, table, B, K, N)
    out = _tc_transpose(pm, B, N, D)
    return out.reshape(B, D, H, W)


# speedup vs baseline: 11.3686x; 1.0427x over previous
import functools

import jax
import jax.numpy as jnp
from jax import lax
from jax.experimental import pallas as pl
from jax.experimental.pallas import tpu as pltpu
from jax.experimental.pallas import tpu_sc as plsc

NUM_WORKERS = 32
CHUNK = 128
LANES = 16


@functools.partial(jax.jit, static_argnums=(2, 3, 4))
def _sc_embed(idx, table, B, K, N):
    D = table.shape[1]
    chunks_per_batch = N // CHUNK
    batches_per_worker = B // NUM_WORKERS
    total_chunks = batches_per_worker * chunks_per_batch

    mesh = plsc.VectorSubcoreMesh(core_axis_name="c", subcore_axis_name="s")

    @functools.partial(
        pl.kernel,
        out_type=jax.ShapeDtypeStruct((B, N, D), jnp.float32),
        mesh=mesh,
        scratch_types=[
            pltpu.VMEM((K, CHUNK), jnp.int32),
            pltpu.VMEM((CHUNK, D), jnp.float32),
            pltpu.SemaphoreType.DMA,
        ],
        compiler_params=pltpu.CompilerParams(use_tc_tiling_on_sc=False),
    )
    def body(idx_hbm, table_hbm, out_hbm, idx_v, acc_t, sem):
        wid = lax.axis_index("s") * 2 + lax.axis_index("c")

        def chunk_step(i, _):
            b = wid * batches_per_worker + (i // chunks_per_batch)
            base = (i % chunks_per_batch) * CHUNK
            pltpu.sync_copy(idx_hbm.at[b, :, pl.ds(base, CHUNK)], idx_v)
            pltpu.async_copy(table_hbm.at[idx_v.at[0]], acc_t, sem).wait()
            copies = [
                pltpu.async_copy(table_hbm.at[idx_v.at[k]], acc_t, sem,
                                 add=True)
                for k in range(1, K)
            ]
            for cp in copies:
                cp.wait()
            pltpu.sync_copy(acc_t, out_hbm.at[b, pl.ds(base, CHUNK), :])
            return 0

        lax.fori_loop(0, total_chunks, chunk_step, 0, unroll=False)

    return body(idx, table)


def _transpose_block(scale, x_ref, o_ref):
    o_ref[...] = jnp.swapaxes(x_ref[...], 1, 2) * scale


@functools.partial(jax.jit, static_argnums=(1, 2, 3, 4))
def _tc_transpose(x, B, N, D, scale):
    NB = 512
    return pl.pallas_call(
        functools.partial(_transpose_block, scale),
        grid=(B, N // NB),
        in_specs=[pl.BlockSpec((1, NB, D), lambda b, n: (b, n, 0))],
        out_specs=pl.BlockSpec((1, D, NB), lambda b, n: (b, 0, n)),
        out_shape=jax.ShapeDtypeStruct((B, D, N), jnp.float32),
    )(x)


def kernel(inputs, table):
    B, K, H, W = inputs.shape
    N = H * W
    D = table.shape[1]
    idx = inputs.reshape(B, K, N).astype(jnp.int32)
    pm = _sc_embed(idx, table, B, K, N)
    out = _tc_transpose(pm, B, N, D, 1.0 / K)
    return out.reshape(B, D, H, W)
